# baseline (speedup 1.0000x reference)
import jax
import jax.numpy as jnp
from jax import lax
from jax.experimental import pallas as pl
from jax.experimental.pallas import tpu as pltpu
from jax.experimental.pallas import tpu_sc as plsc

N_EDGES = 160000
D_FEAT = 256
N_SEG = 10000
NC = 2
NS = 16
NW = NC * NS
W_MAX = 340
ACC_ROWS = 344
CHUNK = 80
IB = 16
IDXB = CHUNK * IB
LANES = 16
BOUNDS_LEN = 9 * LANES


def _seg_sum_body(x_hbm, idx_hbm, bounds_hbm, out_hbm, bounds_v,
                  rows_v0, rows_v1, idx_v, acc_v, sem0, sem1):
    c = lax.axis_index("c")
    s = lax.axis_index("s")
    wid = c * NS + s

    pltpu.sync_copy(bounds_hbm, bounds_v)
    elo = bounds_v[pl.ds(wid, LANES)][0]
    ehi = bounds_v[pl.ds(NW + wid, LANES)][0]
    glo = bounds_v[pl.ds(2 * NW + wid, LANES)][0]
    ghi = bounds_v[pl.ds(3 * NW + wid, LANES)][0]
    nrows = ghi - glo
    nchunks = (ehi - elo) // CHUNK

    rows = (rows_v0, rows_v1)
    sems = (sem0, sem1)

    def start(b, ci):
        st = pl.multiple_of(elo + ci * CHUNK, CHUNK)
        pltpu.async_copy(x_hbm.at[pl.ds(st, CHUNK)], rows[b], sems[b])

    def wait(b, ci):
        st = pl.multiple_of(elo + ci * CHUNK, CHUNK)
        pltpu.make_async_copy(x_hbm.at[pl.ds(st, CHUNK)],
                              rows[b], sems[b]).wait()

    @pl.when(nchunks > 0)
    def _prime0():
        start(0, 0)

    @pl.when(nchunks > 1)
    def _prime1():
        start(1, 1)

    zeros16 = jnp.zeros((LANES,), jnp.float32)

    def zero_body(i, carry):
        base = pl.multiple_of(i * LANES * LANES, LANES)
        for u in range(LANES):
            acc_v[pl.ds(base + u * LANES, LANES)] = zeros16
        return carry

    lax.fori_loop(0, ACC_ROWS * D_FEAT // (LANES * LANES), zero_body, 0)

    NG = D_FEAT // LANES

    def flush(cur, accs):
        ab = cur * D_FEAT
        for g in range(NG):
            plsc.addupdate(acc_v.at[pl.ds(ab + g * LANES, LANES)], accs[g])

    def compute(b, ci, carry):
        rv = rows[b]
        livei = (ci < nchunks).astype(jnp.int32)
        dead_off = (1 - livei) * W_MAX
        sub = ci % IB

        @pl.when((sub == 0) & (ci < nchunks))
        def _stage_idx():
            blk = pl.multiple_of(elo + (ci // IB) * IDXB, CHUNK)
            pltpu.sync_copy(idx_hbm.at[pl.ds(blk, IDXB)], idx_v)

        ibase = pl.multiple_of(sub * CHUNK, LANES)

        def grp_body(j, carry2):
            cur = carry2[0]
            accs = carry2[1:]
            jb = pl.multiple_of(j * LANES, LANES)
            v = idx_v[pl.ds(ibase + jb, LANES)]
            local = v - glo
            ok = (local >= 0) & (local < nrows)
            wvec = jnp.where(ok, local, W_MAX)
            wvec = wvec * livei + dead_off
            for l in range(LANES):
                row = wvec[l]
                same = row == cur
                keep = same.astype(jnp.float32)

                @pl.when(jnp.logical_not(same))
                def _do_flush(cur=cur, accs=accs):
                    flush(cur, accs)

                e = jb + l
                vals = [rv[e, pl.ds(g * LANES, LANES)] for g in range(NG)]
                accs = tuple(accs[g] * keep + vals[g] for g in range(NG))
                cur = row
            return (cur,) + accs

        return lax.fori_loop(0, CHUNK // LANES, grp_body, carry)

    def outer(i2, carry):
        for b in range(2):
            ci = i2 * 2 + b

            @pl.when(ci < nchunks)
            def _wait(b=b, ci=ci):
                wait(b, ci)

            carry = compute(b, ci, carry)

            @pl.when(ci + 2 < nchunks)
            def _next(b=b, ci=ci):
                start(b, ci + 2)

        return carry

    zero_row = jnp.zeros((LANES,), jnp.float32)
    carry0 = (jnp.int32(W_MAX),) + tuple(zero_row for _ in range(NG))
    final = lax.fori_loop(0, (nchunks + 1) // 2, outer, carry0)
    flush(final[0], final[1:])

    def out_body(i, carry):
        pltpu.sync_copy(
            acc_v.at[pl.ds(i * LANES * D_FEAT, LANES * D_FEAT)],
            out_hbm.at[pl.ds((glo + i * LANES) * D_FEAT, LANES * D_FEAT)])
        return carry

    n_full = nrows // LANES
    lax.fori_loop(0, n_full, out_body, 0)
    off = n_full * LANES
    for sub in (8, 4, 2, 1):
        take = (nrows // sub) % 2

        @pl.when(take == 1)
        def _copy_rem(off=off, sub=sub):
            o = pl.multiple_of(off * D_FEAT, 8)
            pltpu.sync_copy(
                acc_v.at[pl.ds(o, sub * D_FEAT)],
                out_hbm.at[pl.ds((glo + off) * D_FEAT, sub * D_FEAT)])

        off = off + take * sub


def _seg_sum_sc(x_flat, idx32, bounds):
    mesh = plsc.VectorSubcoreMesh(core_axis_name="c", subcore_axis_name="s")
    k = pl.kernel(
        _seg_sum_body,
        out_type=jax.ShapeDtypeStruct((N_SEG * D_FEAT,), jnp.float32),
        mesh=mesh,
        scratch_types=[
            pltpu.VMEM((BOUNDS_LEN,), jnp.int32),
            pltpu.VMEM((CHUNK, D_FEAT), jnp.float32),
            pltpu.VMEM((CHUNK, D_FEAT), jnp.float32),
            pltpu.VMEM((IDXB,), jnp.int32),
            pltpu.VMEM((ACC_ROWS * D_FEAT,), jnp.float32),
            pltpu.SemaphoreType.DMA,
            pltpu.SemaphoreType.DMA,
        ],
    )
    return k(x_flat, idx32, bounds)


def kernel(x, index, dim_size):
    del dim_size
    idx32 = index.astype(jnp.int32)

    wq = jnp.arange(1, NW, dtype=jnp.int32) * (N_EDGES // NW)
    ids = jnp.arange(NW + 1, dtype=jnp.int32)
    q = jnp.concatenate([jnp.zeros((1,), jnp.int32), idx32[wq],
                         jnp.full((1,), N_SEG, jnp.int32)])
    lbound = N_SEG - W_MAX * (NW - ids)
    cvals = jnp.maximum(q, lbound)
    b = cvals - W_MAX * ids
    tri = ids[:, None] >= ids[None, :]
    big = jnp.int32(2 ** 30)
    g = jnp.min(jnp.where(tri, b[None, :], big), axis=1) + W_MAX * ids

    sedge = jnp.searchsorted(idx32, g, method="compare_all").astype(jnp.int32)
    elo = sedge[:NW] // CHUNK * CHUNK
    ehi = (sedge[1:] + CHUNK - 1) // CHUNK * CHUNK
    bounds = jnp.concatenate([
        elo, ehi, g[:NW], g[1:],
        jnp.zeros((LANES,), jnp.int32),
    ])

    idx_pad = jnp.concatenate([idx32, jnp.zeros((IDXB,), jnp.int32)])
    out = _seg_sum_sc(x, idx_pad, bounds)
    return out.reshape(N_SEG, D_FEAT)


# speedup vs baseline: 1.0188x; 1.0188x over previous
import jax
import jax.numpy as jnp
from jax import lax
from jax.experimental import pallas as pl
from jax.experimental.pallas import tpu as pltpu
from jax.experimental.pallas import tpu_sc as plsc

N_EDGES = 160000
D_FEAT = 256
N_SEG = 10000
NC = 2
NS = 16
NW = NC * NS
W_MAX = 360
ACC_ROWS = 364
CHUNK = 64
LANES = 16
BOUNDS_LEN = 9 * LANES


def _seg_sum_body(x_hbm, idx_hbm, bounds_hbm, out_hbm, bounds_v,
                  rows_v0, rows_v1, idx_v0, idx_v1, acc_v, sem0, sem1):
    c = lax.axis_index("c")
    s = lax.axis_index("s")
    wid = c * NS + s

    pltpu.sync_copy(bounds_hbm, bounds_v)
    elo = bounds_v[pl.ds(wid, LANES)][0]
    ehi = bounds_v[pl.ds(NW + wid, LANES)][0]
    glo = bounds_v[pl.ds(2 * NW + wid, LANES)][0]
    ghi = bounds_v[pl.ds(3 * NW + wid, LANES)][0]
    nrows = ghi - glo
    nchunks = (ehi - elo) // CHUNK

    rows = (rows_v0, rows_v1)
    idxs = (idx_v0, idx_v1)
    sems = (sem0, sem1)

    def start(b, ci):
        st = pl.multiple_of(elo + ci * CHUNK, CHUNK)
        pltpu.async_copy(x_hbm.at[pl.ds(st, CHUNK)], rows[b], sems[b])
        pltpu.async_copy(idx_hbm.at[pl.ds(st, CHUNK)], idxs[b], sems[b])

    def wait(b, ci):
        st = pl.multiple_of(elo + ci * CHUNK, CHUNK)
        pltpu.make_async_copy(x_hbm.at[pl.ds(st, CHUNK)],
                              rows[b], sems[b]).wait()
        pltpu.make_async_copy(idx_hbm.at[pl.ds(st, CHUNK)],
                              idxs[b], sems[b]).wait()

    @pl.when(nchunks > 0)
    def _prime0():
        start(0, 0)

    @pl.when(nchunks > 1)
    def _prime1():
        start(1, 1)

    zeros16 = jnp.zeros((LANES,), jnp.float32)

    def zero_body(i, carry):
        base = pl.multiple_of(i * LANES * LANES, LANES)
        for u in range(LANES):
            acc_v[pl.ds(base + u * LANES, LANES)] = zeros16
        return carry

    lax.fori_loop(0, ACC_ROWS * D_FEAT // (LANES * LANES), zero_body, 0)

    NG = D_FEAT // LANES

    def flush(cur, accs):
        ab = cur * D_FEAT
        for g in range(NG):
            plsc.addupdate(acc_v.at[pl.ds(ab + g * LANES, LANES)], accs[g])

    def compute(b, ci, carry):
        rv, iv = rows[b], idxs[b]
        livei = (ci < nchunks).astype(jnp.int32)
        dead_off = (1 - livei) * W_MAX

        def grp_body(j, carry2):
            cur = carry2[0]
            accs = carry2[1:]
            jb = pl.multiple_of(j * LANES, LANES)
            v = iv[pl.ds(jb, LANES)]
            local = v - glo
            ok = (local >= 0) & (local < nrows)
            wvec = jnp.where(ok, local, W_MAX)
            wvec = wvec * livei + dead_off
            for l in range(LANES):
                row = wvec[l]
                same = row == cur
                keep = same.astype(jnp.float32)

                @pl.when(jnp.logical_not(same))
                def _do_flush(cur=cur, accs=accs):
                    flush(cur, accs)

                e = jb + l
                vals = [rv[e, pl.ds(g * LANES, LANES)] for g in range(NG)]
                accs = tuple(accs[g] * keep + vals[g] for g in range(NG))
                cur = row
            return (cur,) + accs

        return lax.fori_loop(0, CHUNK // LANES, grp_body, carry)

    def outer(i2, carry):
        for b in range(2):
            ci = i2 * 2 + b

            @pl.when(ci < nchunks)
            def _wait(b=b, ci=ci):
                wait(b, ci)

            carry = compute(b, ci, carry)

            @pl.when(ci + 2 < nchunks)
            def _next(b=b, ci=ci):
                start(b, ci + 2)

        return carry

    zero_row = jnp.zeros((LANES,), jnp.float32)
    carry0 = (jnp.int32(W_MAX),) + tuple(zero_row for _ in range(NG))
    final = lax.fori_loop(0, (nchunks + 1) // 2, outer, carry0)
    flush(final[0], final[1:])

    def out_body(i, carry):
        pltpu.sync_copy(
            acc_v.at[pl.ds(i * LANES * D_FEAT, LANES * D_FEAT)],
            out_hbm.at[pl.ds((glo + i * LANES) * D_FEAT, LANES * D_FEAT)])
        return carry

    n_full = nrows // LANES
    lax.fori_loop(0, n_full, out_body, 0)
    off = n_full * LANES
    for sub in (8, 4, 2, 1):
        take = (nrows // sub) % 2

        @pl.when(take == 1)
        def _copy_rem(off=off, sub=sub):
            o = pl.multiple_of(off * D_FEAT, 8)
            pltpu.sync_copy(
                acc_v.at[pl.ds(o, sub * D_FEAT)],
                out_hbm.at[pl.ds((glo + off) * D_FEAT, sub * D_FEAT)])

        off = off + take * sub


def _seg_sum_sc(x_flat, idx32, bounds):
    mesh = plsc.VectorSubcoreMesh(core_axis_name="c", subcore_axis_name="s")
    k = pl.kernel(
        _seg_sum_body,
        out_type=jax.ShapeDtypeStruct((N_SEG * D_FEAT,), jnp.float32),
        mesh=mesh,
        scratch_types=[
            pltpu.VMEM((BOUNDS_LEN,), jnp.int32),
            pltpu.VMEM((CHUNK, D_FEAT), jnp.float32),
            pltpu.VMEM((CHUNK, D_FEAT), jnp.float32),
            pltpu.VMEM((CHUNK,), jnp.int32),
            pltpu.VMEM((CHUNK,), jnp.int32),
            pltpu.VMEM((ACC_ROWS * D_FEAT,), jnp.float32),
            pltpu.SemaphoreType.DMA,
            pltpu.SemaphoreType.DMA,
        ],
    )
    return k(x_flat, idx32, bounds)


def kernel(x, index, dim_size):
    del dim_size
    idx32 = index.astype(jnp.int32)

    wq = jnp.arange(1, NW, dtype=jnp.int32) * (N_EDGES // NW)
    ids = jnp.arange(NW + 1, dtype=jnp.int32)
    q = jnp.concatenate([jnp.zeros((1,), jnp.int32), idx32[wq],
                         jnp.full((1,), N_SEG, jnp.int32)])
    lbound = N_SEG - W_MAX * (NW - ids)
    cvals = jnp.maximum(q, lbound)
    b = cvals - W_MAX * ids
    tri = ids[:, None] >= ids[None, :]
    big = jnp.int32(2 ** 30)
    g = jnp.min(jnp.where(tri, b[None, :], big), axis=1) + W_MAX * ids

    sedge = jnp.searchsorted(idx32, g, method="compare_all").astype(jnp.int32)
    elo = sedge[:NW] // CHUNK * CHUNK
    ehi = (sedge[1:] + CHUNK - 1) // CHUNK * CHUNK
    bounds = jnp.concatenate([
        elo, ehi, g[:NW], g[1:],
        jnp.zeros((LANES,), jnp.int32),
    ])

    out = _seg_sum_sc(x, idx32, bounds)
    return out.reshape(N_SEG, D_FEAT)
